# Initial kernel scaffold; baseline (speedup 1.0000x reference)
import jax
import jax.numpy as jnp
from jax.experimental import pallas as pl


def kernel(value_embed, all_weights, weight_embedding):
    raise NotImplementedError("write your pallas kernel here")


# speedup vs baseline: 3.4296x; 3.4296x over previous
import jax
import jax.numpy as jnp
from jax.experimental import pallas as pl
from jax.experimental.pallas import tpu as pltpu

_MAX_WEIGHT = 20
_HIDDEN = 64
_BR = 4096
_TPAD = 32


def _tc_body(w_ref, v_ref, t_ref, o_ref):
    w = w_ref[0, 0, :]
    wids = jnp.clip(jnp.minimum(w, _MAX_WEIGHT + 1) - 1, 0, _MAX_WEIGHT)
    one_hot = (wids[:, None] == jax.lax.broadcasted_iota(jnp.int32, (1, _TPAD), 1)
               ).astype(jnp.float32)
    w_embed = jax.lax.dot_general(
        one_hot, t_ref[...],
        dimension_numbers=(((1,), (0,)), ((), ())),
        preferred_element_type=jnp.float32,
    )
    o_ref[...] = v_ref[...] + w_embed


def kernel(value_embed, all_weights, weight_embedding):
    n, hidden = value_embed.shape
    grid = n // _BR
    w3d = all_weights.reshape(grid, 1, _BR)
    table = jnp.zeros((_TPAD, hidden), jnp.float32).at[:_MAX_WEIGHT + 1].set(
        weight_embedding)
    return pl.pallas_call(
        _tc_body,
        grid=(grid,),
        in_specs=[
            pl.BlockSpec((1, 1, _BR), lambda i: (i, 0, 0)),
            pl.BlockSpec((_BR, hidden), lambda i: (i, 0)),
            pl.BlockSpec((_TPAD, hidden), lambda i: (0, 0)),
        ],
        out_specs=pl.BlockSpec((_BR, hidden), lambda i: (i, 0)),
        out_shape=jax.ShapeDtypeStruct((n, hidden), jnp.float32),
        compiler_params=pltpu.CompilerParams(
            dimension_semantics=("parallel",),
        ),
    )(w3d, value_embed, table)


# speedup vs baseline: 3.6975x; 1.0781x over previous
import jax
import jax.numpy as jnp
from jax.experimental import pallas as pl
from jax.experimental.pallas import tpu as pltpu

_MAX_WEIGHT = 20
_HIDDEN = 64
_BR = 16384
_TPAD = 32


def _tc_body(w_ref, v_ref, t_ref, o_ref):
    w = w_ref[0, 0, :]
    wids = jnp.clip(jnp.minimum(w, _MAX_WEIGHT + 1) - 1, 0, _MAX_WEIGHT)
    one_hot = (wids[:, None] == jax.lax.broadcasted_iota(jnp.int32, (1, _TPAD), 1)
               ).astype(jnp.float32)
    w_embed = jax.lax.dot_general(
        one_hot, t_ref[...],
        dimension_numbers=(((1,), (0,)), ((), ())),
        preferred_element_type=jnp.float32,
    )
    o_ref[...] = v_ref[...] + w_embed


def kernel(value_embed, all_weights, weight_embedding):
    n, hidden = value_embed.shape
    grid = n // _BR
    w3d = all_weights.reshape(grid, 1, _BR)
    table = jnp.zeros((_TPAD, hidden), jnp.float32).at[:_MAX_WEIGHT + 1].set(
        weight_embedding)
    return pl.pallas_call(
        _tc_body,
        grid=(grid,),
        in_specs=[
            pl.BlockSpec((1, 1, _BR), lambda i: (i, 0, 0)),
            pl.BlockSpec((_BR, hidden), lambda i: (i, 0)),
            pl.BlockSpec((_TPAD, hidden), lambda i: (0, 0)),
        ],
        out_specs=pl.BlockSpec((_BR, hidden), lambda i: (i, 0)),
        out_shape=jax.ShapeDtypeStruct((n, hidden), jnp.float32),
        compiler_params=pltpu.CompilerParams(
            dimension_semantics=("parallel",),
        ),
    )(w3d, value_embed, table)
